# baseline (speedup 1.0000x reference)
import functools

import jax
import jax.numpy as jnp
from jax.experimental import pallas as pl
from jax.experimental.pallas import tpu as pltpu


def _se_kernel4d(x_ref, w1_ref, w2_ref, o_ref, *, inv_hw):
    xb = x_ref[0]
    colsum = jnp.sum(xb, axis=1, dtype=jnp.float32)
    pooled = jnp.sum(colsum, axis=1, keepdims=True,
                     dtype=jnp.float32) * inv_hw
    h = jnp.dot(w1_ref[...], pooled, preferred_element_type=jnp.float32)
    h = jnp.maximum(h, 0.0)
    s = jax.nn.sigmoid(
        jnp.dot(w2_ref[...], h, preferred_element_type=jnp.float32))
    o_ref[0] = xb * s[:, :, None].astype(o_ref.dtype)


def kernel(x, w1, w2):
    B, C, H, W = x.shape
    mid = w1.shape[0]
    itemsize = jnp.dtype(x.dtype).itemsize

    cost = pl.CostEstimate(
        flops=2 * B * C * H * W + 4 * B * C * mid,
        transcendentals=B * C,
        bytes_accessed=2 * B * C * H * W * itemsize
        + (w1.size + w2.size) * itemsize,
    )
    return pl.pallas_call(
        functools.partial(_se_kernel4d, inv_hw=1.0 / (H * W)),
        out_shape=jax.ShapeDtypeStruct((B, C, H, W), x.dtype),
        grid=(B,),
        in_specs=[
            pl.BlockSpec((1, C, H, W), lambda b: (b, 0, 0, 0)),
            pl.BlockSpec((mid, C), lambda b: (0, 0)),
            pl.BlockSpec((C, mid), lambda b: (0, 0)),
        ],
        out_specs=pl.BlockSpec((1, C, H, W), lambda b: (b, 0, 0, 0)),
        compiler_params=pltpu.CompilerParams(
            dimension_semantics=("parallel",),
            vmem_limit_bytes=64 << 20,
        ),
        cost_estimate=cost,
    )(x, w1, w2)


# speedup vs baseline: 6.7303x; 6.7303x over previous
import functools

import jax
import jax.numpy as jnp
from jax.experimental import pallas as pl
from jax.experimental.pallas import tpu as pltpu


def _se_nhwc_kernel(x_ref, w1t_ref, w2t_ref, o_ref, *, inv_hw):
    xb = x_ref[0]
    rowsum = jnp.sum(xb, axis=0, dtype=jnp.float32)
    pooled = jnp.sum(rowsum, axis=0, keepdims=True,
                     dtype=jnp.float32) * inv_hw
    h = jnp.dot(pooled, w1t_ref[...], preferred_element_type=jnp.float32)
    h = jnp.maximum(h, 0.0)
    s = jax.nn.sigmoid(
        jnp.dot(h, w2t_ref[...], preferred_element_type=jnp.float32))
    o_ref[0] = xb * s[None].astype(o_ref.dtype)


def kernel(x, w1, w2):
    B, C, H, W = x.shape
    mid = w1.shape[0]
    itemsize = jnp.dtype(x.dtype).itemsize

    xt = jnp.transpose(x, (0, 2, 3, 1))
    w1t = jnp.transpose(w1)
    w2t = jnp.transpose(w2)

    cost = pl.CostEstimate(
        flops=2 * B * C * H * W + 4 * B * C * mid,
        transcendentals=B * C,
        bytes_accessed=2 * B * C * H * W * itemsize
        + (w1.size + w2.size) * itemsize,
    )
    out = pl.pallas_call(
        functools.partial(_se_nhwc_kernel, inv_hw=1.0 / (H * W)),
        out_shape=jax.ShapeDtypeStruct((B, H, W, C), x.dtype),
        grid=(B,),
        in_specs=[
            pl.BlockSpec((1, H, W, C), lambda b: (b, 0, 0, 0)),
            pl.BlockSpec((C, mid), lambda b: (0, 0)),
            pl.BlockSpec((mid, C), lambda b: (0, 0)),
        ],
        out_specs=pl.BlockSpec((1, H, W, C), lambda b: (b, 0, 0, 0)),
        compiler_params=pltpu.CompilerParams(
            dimension_semantics=("parallel",),
            vmem_limit_bytes=64 << 20,
        ),
        cost_estimate=cost,
    )(xt, w1t, w2t)
    return jnp.transpose(out, (0, 3, 1, 2))


# speedup vs baseline: 7.0570x; 1.0485x over previous
import functools

import jax
import jax.numpy as jnp
from jax.experimental import pallas as pl
from jax.experimental.pallas import tpu as pltpu


def _se_nhwc_kernel(x_ref, w1t_ref, w2t_ref, o_ref, *, inv_hw):
    xb = x_ref[...]
    rowsum = jnp.sum(xb, axis=1, dtype=jnp.float32)
    pooled = jnp.sum(rowsum, axis=1, dtype=jnp.float32) * inv_hw
    h = jnp.dot(pooled, w1t_ref[...], preferred_element_type=jnp.float32)
    h = jnp.maximum(h, 0.0)
    s = jax.nn.sigmoid(
        jnp.dot(h, w2t_ref[...], preferred_element_type=jnp.float32))
    o_ref[...] = xb * s[:, None, None, :].astype(o_ref.dtype)


def kernel(x, w1, w2):
    B, C, H, W = x.shape
    mid = w1.shape[0]
    itemsize = jnp.dtype(x.dtype).itemsize

    xt = jnp.transpose(x, (0, 2, 3, 1))
    w1t = jnp.transpose(w1)
    w2t = jnp.transpose(w2)

    cost = pl.CostEstimate(
        flops=2 * B * C * H * W + 4 * B * C * mid,
        transcendentals=B * C,
        bytes_accessed=2 * B * C * H * W * itemsize
        + (w1.size + w2.size) * itemsize,
    )
    bt = 2
    out = pl.pallas_call(
        functools.partial(_se_nhwc_kernel, inv_hw=1.0 / (H * W)),
        out_shape=jax.ShapeDtypeStruct((B, H, W, C), x.dtype),
        grid=(B // bt,),
        in_specs=[
            pl.BlockSpec((bt, H, W, C), lambda b: (b, 0, 0, 0)),
            pl.BlockSpec((C, mid), lambda b: (0, 0)),
            pl.BlockSpec((mid, C), lambda b: (0, 0)),
        ],
        out_specs=pl.BlockSpec((bt, H, W, C), lambda b: (b, 0, 0, 0)),
        compiler_params=pltpu.CompilerParams(
            dimension_semantics=("parallel",),
            vmem_limit_bytes=64 << 20,
        ),
        cost_estimate=cost,
    )(xt, w1t, w2t)
    return jnp.transpose(out, (0, 3, 1, 2))


# speedup vs baseline: 7.1434x; 1.0122x over previous
import functools

import jax
import jax.numpy as jnp
from jax.experimental import pallas as pl
from jax.experimental.pallas import tpu as pltpu


def _se_nhwc_kernel(x_ref, w1t_ref, w2t_ref, o_ref, *, inv_hw):
    xb = x_ref[...]
    rowsum = jnp.sum(xb, axis=1, dtype=jnp.float32)
    pooled = jnp.sum(rowsum, axis=1, dtype=jnp.float32) * inv_hw
    h = jnp.dot(pooled, w1t_ref[...], preferred_element_type=jnp.float32)
    h = jnp.maximum(h, 0.0)
    s = jax.nn.sigmoid(
        jnp.dot(h, w2t_ref[...], preferred_element_type=jnp.float32))
    o_ref[...] = xb * s[:, None, None, :].astype(o_ref.dtype)


def kernel(x, w1, w2):
    B, C, H, W = x.shape
    mid = w1.shape[0]
    itemsize = jnp.dtype(x.dtype).itemsize

    xt = jnp.transpose(x, (0, 2, 3, 1))
    w1t = jnp.transpose(w1)
    w2t = jnp.transpose(w2)

    cost = pl.CostEstimate(
        flops=2 * B * C * H * W + 4 * B * C * mid,
        transcendentals=B * C,
        bytes_accessed=2 * B * C * H * W * itemsize
        + (w1.size + w2.size) * itemsize,
    )
    bt = 4
    out = pl.pallas_call(
        functools.partial(_se_nhwc_kernel, inv_hw=1.0 / (H * W)),
        out_shape=jax.ShapeDtypeStruct((B, H, W, C), x.dtype),
        grid=(B // bt,),
        in_specs=[
            pl.BlockSpec((bt, H, W, C), lambda b: (b, 0, 0, 0)),
            pl.BlockSpec((C, mid), lambda b: (0, 0)),
            pl.BlockSpec((mid, C), lambda b: (0, 0)),
        ],
        out_specs=pl.BlockSpec((bt, H, W, C), lambda b: (b, 0, 0, 0)),
        compiler_params=pltpu.CompilerParams(
            dimension_semantics=("parallel",),
            vmem_limit_bytes=64 << 20,
        ),
        cost_estimate=cost,
    )(xt, w1t, w2t)
    return jnp.transpose(out, (0, 3, 1, 2))


# speedup vs baseline: 7.1514x; 1.0011x over previous
import functools

import jax
import jax.numpy as jnp
from jax.experimental import pallas as pl
from jax.experimental.pallas import tpu as pltpu


def _se_nhwc_kernel(x_ref, w1t_ref, w2t_ref, o_ref, *, inv_hw):
    xb = x_ref[...]
    rowsum = jnp.sum(xb, axis=1, dtype=jnp.float32)
    pooled = jnp.sum(rowsum, axis=1, dtype=jnp.float32) * inv_hw
    h = jnp.dot(pooled, w1t_ref[...], preferred_element_type=jnp.float32)
    h = jnp.maximum(h, 0.0)
    s = jax.nn.sigmoid(
        jnp.dot(h, w2t_ref[...], preferred_element_type=jnp.float32))
    o_ref[...] = xb * s[:, None, None, :].astype(o_ref.dtype)


def kernel(x, w1, w2):
    B, C, H, W = x.shape
    mid = w1.shape[0]
    itemsize = jnp.dtype(x.dtype).itemsize

    xt = jnp.transpose(x, (0, 2, 3, 1))
    w1t = jnp.transpose(w1)
    w2t = jnp.transpose(w2)

    cost = pl.CostEstimate(
        flops=2 * B * C * H * W + 4 * B * C * mid,
        transcendentals=B * C,
        bytes_accessed=2 * B * C * H * W * itemsize
        + (w1.size + w2.size) * itemsize,
    )
    bt = next(b for b in (4, 2, 1) if B % b == 0)
    out = pl.pallas_call(
        functools.partial(_se_nhwc_kernel, inv_hw=1.0 / (H * W)),
        out_shape=jax.ShapeDtypeStruct((B, H, W, C), x.dtype),
        grid=(B // bt,),
        in_specs=[
            pl.BlockSpec((bt, H, W, C), lambda b: (b, 0, 0, 0)),
            pl.BlockSpec((C, mid), lambda b: (0, 0)),
            pl.BlockSpec((mid, C), lambda b: (0, 0)),
        ],
        out_specs=pl.BlockSpec((bt, H, W, C), lambda b: (b, 0, 0, 0)),
        compiler_params=pltpu.CompilerParams(
            dimension_semantics=("parallel",),
            vmem_limit_bytes=64 << 20,
        ),
        cost_estimate=cost,
    )(xt, w1t, w2t)
    return jnp.transpose(out, (0, 3, 1, 2))
